# baseline (speedup 1.0000x reference)
import functools

import jax
import jax.numpy as jnp
from jax import lax
from jax.experimental import pallas as pl
from jax.experimental.pallas import tpu as pltpu
from jax.experimental.pallas import tpu_sc as plsc

N_NODES = 10000
N_EDGES = 320000
D_IN = 128
D_HID = 128
N_CLASSES = 64

NPAD = 10240
NC = 2
NS = 16
NW = NC * NS
EPW = 10240
CH = 128
BLK = 2560
RPT = NPAD // NS

_MESH = plsc.VectorSubcoreMesh(
    core_axis_name="c", subcore_axis_name="s", num_cores=NC, num_subcores=NS
)

_F32 = jnp.float32


def _zv():
    return jnp.zeros((16,), _F32)


def _deg_body(src_hbm, dst_hbm, out_hbm, is0, is1, id0, id1, ones_v, zbuf,
              deg_s, deg_d, ss0, ss1, sd0, sd1):
    c = lax.axis_index("c")
    s = lax.axis_index("s")
    for j in range(CH // 16):
        ones_v[pl.ds(j * 16, 16)] = jnp.ones((16,), _F32)
    for j in range(RPT // 16):
        zbuf[pl.ds(j * 16, 16)] = _zv()
    zoff = pl.multiple_of(s * RPT, 8)
    pltpu.sync_copy(zbuf, deg_s.at[pl.ds(zoff, RPT)])
    pltpu.sync_copy(zbuf, deg_d.at[pl.ds(zoff, RPT)])
    plsc.subcore_barrier()
    ebase = pl.multiple_of((c * NS + s) * EPW, 8)

    pltpu.async_copy(src_hbm.at[pl.ds(ebase, CH)], is0, ss0)
    pltpu.async_copy(dst_hbm.at[pl.ds(ebase, CH)], id0, sd0)
    pltpu.async_copy(src_hbm.at[pl.ds(pl.multiple_of(ebase + CH, 8), CH)], is1, ss1)
    pltpu.async_copy(dst_hbm.at[pl.ds(pl.multiple_of(ebase + CH, 8), CH)], id1, sd1)

    def body(i, carry):
        a = 2 * i
        n0 = pl.multiple_of(ebase + ((a + 2) % NCHUNK) * CH, 8)
        n1 = pl.multiple_of(ebase + ((a + 3) % NCHUNK) * CH, 8)
        pltpu.make_async_copy(src_hbm.at[pl.ds(ebase, CH)], is0, ss0).wait()
        pltpu.sync_copy(ones_v, deg_s.at[is0], add=True)
        pltpu.async_copy(src_hbm.at[pl.ds(n0, CH)], is0, ss0)
        pltpu.make_async_copy(dst_hbm.at[pl.ds(ebase, CH)], id0, sd0).wait()
        pltpu.sync_copy(ones_v, deg_d.at[id0], add=True)
        pltpu.async_copy(dst_hbm.at[pl.ds(n0, CH)], id0, sd0)
        pltpu.make_async_copy(src_hbm.at[pl.ds(ebase, CH)], is1, ss1).wait()
        pltpu.sync_copy(ones_v, deg_s.at[is1], add=True)
        pltpu.async_copy(src_hbm.at[pl.ds(n1, CH)], is1, ss1)
        pltpu.make_async_copy(dst_hbm.at[pl.ds(ebase, CH)], id1, sd1).wait()
        pltpu.sync_copy(ones_v, deg_d.at[id1], add=True)
        pltpu.async_copy(dst_hbm.at[pl.ds(n1, CH)], id1, sd1)
        return carry

    lax.fori_loop(0, nchunk // 2, body, 0)
    pltpu.make_async_copy(src_hbm.at[pl.ds(ebase, CH)], is0, ss0).wait()
    pltpu.make_async_copy(dst_hbm.at[pl.ds(ebase, CH)], id0, sd0).wait()
    pltpu.make_async_copy(src_hbm.at[pl.ds(ebase, CH)], is1, ss1).wait()
    pltpu.make_async_copy(dst_hbm.at[pl.ds(ebase, CH)], id1, sd1).wait()
    plsc.subcore_barrier()
    pltpu.sync_copy(deg_s.at[pl.ds(zoff, RPT)], out_hbm.at[c, 0, pl.ds(zoff, RPT)])
    pltpu.sync_copy(deg_d.at[pl.ds(zoff, RPT)], out_hbm.at[c, 1, pl.ds(zoff, RPT)])


_deg_call = pl.kernel(
    _deg_body,
    out_type=jax.ShapeDtypeStruct((NC, 2, NPAD), _F32),
    mesh=_MESH,
    scratch_types=[
        pltpu.VMEM((CH,), jnp.int32),
        pltpu.VMEM((CH,), jnp.int32),
        pltpu.VMEM((CH,), jnp.int32),
        pltpu.VMEM((CH,), jnp.int32),
        pltpu.VMEM((CH,), _F32),
        pltpu.VMEM((RPT,), _F32),
        pltpu.VMEM_SHARED((NPAD,), _F32),
        pltpu.VMEM_SHARED((NPAD,), _F32),
        pltpu.SemaphoreType.DMA,
        pltpu.SemaphoreType.DMA,
        pltpu.SemaphoreType.DMA,
        pltpu.SemaphoreType.DMA,
    ],
)


def _make_agg(d):
    def _agg_body(p_hbm, src_hbm, dst_hbm, out_hbm, idx_sf,
                  idxb0, idxb1, rows0, rows1, agg_sh,
                  sem0, sem1, ism0, ism1):
        c = lax.axis_index("c")
        s = lax.axis_index("s")
        idxb = [idxb0, idxb1]
        rows = [rows0, rows1]
        sem = [sem0, sem1]
        ism = [ism0, ism1]
        DEPTH = 2

        def zbody(r, carry):
            for k in range(d // 16):
                rows0[r, pl.ds(k * 16, 16)] = _zv()
            return carry

        lax.fori_loop(0, CH, zbody, 0)
        for t in range(RPT // CH):
            roff = pl.multiple_of(s * RPT + t * CH, 8)
            pltpu.sync_copy(rows0, agg_sh.at[pl.ds(roff, CH)])

        tid = c * NS + s
        ebase = pl.multiple_of(tid * EPW, 8)
        nchunk = jnp.minimum(EPW, N_EDGES - tid * EPW) // CH

        def lbody(j, carry):
            off = pl.multiple_of(j * BLK, 8)
            pltpu.sync_copy(
                src_hbm.at[pl.ds(pl.multiple_of(ebase + off, 8), BLK)],
                idx_sf.at[pl.ds(off, BLK)])
            return carry

        lax.fori_loop(0, (nchunk * CH) // BLK, lbody, 0)
        plsc.subcore_barrier()

        for k in range(DEPTH):
            o = pl.multiple_of(k * CH, 8)
            pltpu.async_copy(p_hbm.at[idx_sf.at[pl.ds(o, CH)]], rows[k], sem[k])
            pltpu.async_copy(
                dst_hbm.at[pl.ds(pl.multiple_of(ebase + o, 8), CH)], idxb[k], ism[k])

        def body(i, carry):
            a = DEPTH * i
            for k in range(DEPTH):
                o = pl.multiple_of((a + k) * CH, 8)
                w = jnp.where(a + k + DEPTH >= nchunk,
                              a + k + DEPTH - nchunk, a + k + DEPTH)
                n = pl.multiple_of(w * CH, 8)
                pltpu.make_async_copy(
                    dst_hbm.at[pl.ds(ebase, CH)], idxb[k], ism[k]).wait()
                pltpu.make_async_copy(
                    p_hbm.at[idx_sf.at[pl.ds(o, CH)]], rows[k], sem[k]).wait()
                pltpu.sync_copy(rows[k], agg_sh.at[idxb[k]], add=True)
                pltpu.async_copy(p_hbm.at[idx_sf.at[pl.ds(n, CH)]], rows[k], sem[k])
                pltpu.async_copy(
                    dst_hbm.at[pl.ds(pl.multiple_of(ebase + n, 8), CH)], idxb[k], ism[k])
            return carry

        lax.fori_loop(0, nchunk // DEPTH, body, 0)
        for k in range(DEPTH):
            o = pl.multiple_of(k * CH, 8)
            pltpu.make_async_copy(p_hbm.at[idx_sf.at[pl.ds(o, CH)]], rows[k], sem[k]).wait()
            pltpu.make_async_copy(dst_hbm.at[pl.ds(ebase, CH)], idxb[k], ism[k]).wait()
        plsc.subcore_barrier()
        roff = pl.multiple_of(s * RPT, 8)
        pltpu.sync_copy(agg_sh.at[pl.ds(roff, RPT)], out_hbm.at[c, pl.ds(roff, RPT)])

    return pl.kernel(
        _agg_body,
        out_type=jax.ShapeDtypeStruct((NC, NPAD, d), _F32),
        mesh=_MESH,
        scratch_types=[
            pltpu.VMEM((EPW,), jnp.int32),
            pltpu.VMEM((CH,), jnp.int32),
            pltpu.VMEM((CH,), jnp.int32),
            pltpu.VMEM((CH, d), _F32),
            pltpu.VMEM((CH, d), _F32),
            pltpu.VMEM_SHARED((NPAD, d), _F32),
            pltpu.SemaphoreType.DMA,
            pltpu.SemaphoreType.DMA,
            pltpu.SemaphoreType.DMA,
            pltpu.SemaphoreType.DMA,
        ],
    )


_agg128 = _make_agg(D_HID)

BR = 1024


def _norm(d0, d1):
    deg = d0 + d1
    return lax.rsqrt(jnp.where(deg > 0.0, deg, 1.0))


def _scale_mm_body(x_ref, w_ref, d0_ref, d1_ref, o_ref):
    ns = _norm(d0_ref[...], d1_ref[...])
    o_ref[...] = jnp.dot(x_ref[...], w_ref[...], preferred_element_type=_F32) * ns


def _mid_body(p_ref, dd0_ref, dd1_ref, b1_ref, w2_ref, ds0_ref, ds1_ref, o_ref):
    nd = _norm(dd0_ref[...], dd1_ref[...])
    h = jnp.maximum((p_ref[0] + p_ref[1]) * nd + b1_ref[...], 0.0)
    ns = _norm(ds0_ref[...], ds1_ref[...])
    o_ref[...] = jnp.dot(h, w2_ref[...], preferred_element_type=_F32) * ns


def _fin_body(p_ref, dd0_ref, dd1_ref, b2_ref, o_ref):
    nd = _norm(dd0_ref[...], dd1_ref[...])
    o_ref[...] = (p_ref[0, :, :N_CLASSES] + p_ref[1, :, :N_CLASSES]) * nd + b2_ref[...]


def _col_spec():
    return pl.BlockSpec((BR, 1), lambda i: (i, 0))


def _row_spec(d):
    return pl.BlockSpec((BR, d), lambda i: (i, 0))


def _full_spec(a, b):
    return pl.BlockSpec((a, b), lambda i: (0, 0))


def _scale_mm(x, w, d0, d1):
    d = w.shape[1]
    return pl.pallas_call(
        _scale_mm_body,
        grid=(NPAD // BR,),
        in_specs=[_row_spec(x.shape[1]), _full_spec(*w.shape), _col_spec(), _col_spec()],
        out_specs=_row_spec(d),
        out_shape=jax.ShapeDtypeStruct((NPAD, d), _F32),
    )(x, w, d0, d1)


def _parts_spec(d, br):
    return pl.BlockSpec((NC, br, d), lambda i: (0, i, 0))


def _mid(parts, dd0, dd1, b1, w2, ds0, ds1):
    dout = w2.shape[1]
    return pl.pallas_call(
        _mid_body,
        grid=(NPAD // BR,),
        in_specs=[
            _parts_spec(D_HID, BR), _col_spec(), _col_spec(),
            _full_spec(1, D_HID), _full_spec(D_HID, dout), _col_spec(), _col_spec(),
        ],
        out_specs=_row_spec(dout),
        out_shape=jax.ShapeDtypeStruct((NPAD, dout), _F32),
    )(parts, dd0, dd1, b1, w2, ds0, ds1)


BRF = 1000


def _fin(parts, dd0, dd1, b2):
    return pl.pallas_call(
        _fin_body,
        grid=(N_NODES // BRF,),
        in_specs=[
            _parts_spec(D_HID, BRF),
            pl.BlockSpec((BRF, 1), lambda i: (i, 0)),
            pl.BlockSpec((BRF, 1), lambda i: (i, 0)),
            _full_spec(1, N_CLASSES),
        ],
        out_specs=pl.BlockSpec((BRF, N_CLASSES), lambda i: (i, 0)),
        out_shape=jax.ShapeDtypeStruct((N_NODES, N_CLASSES), _F32),
    )(parts, dd0, dd1, b2)


def kernel(inputs, edge_index, W1, b1, W2, b2):
    x = jnp.pad(inputs, ((0, NPAD - N_NODES), (0, 0)))
    src = edge_index[0]
    dst = edge_index[1]

    deg = _deg_call(src, dst)
    d_s = deg[:, 0, :].reshape(NC, NPAD, 1)
    d_d = deg[:, 1, :].reshape(NC, NPAD, 1)

    p1 = _scale_mm(x, W1, d_s[0], d_s[1])
    parts1 = _agg128(p1, src, dst)
    w2p = jnp.pad(W2, ((0, 0), (0, D_HID - N_CLASSES)))
    p2 = _mid(parts1, d_d[0], d_d[1],
              b1.reshape(1, D_HID), w2p, d_s[0], d_s[1])
    parts2 = _agg128(p2, src, dst)
    return _fin(parts2, d_d[0], d_d[1], b2.reshape(1, N_CLASSES))


# speedup vs baseline: 1.0071x; 1.0071x over previous
import functools

import jax
import jax.numpy as jnp
from jax import lax
from jax.experimental import pallas as pl
from jax.experimental.pallas import tpu as pltpu
from jax.experimental.pallas import tpu_sc as plsc

N_NODES = 10000
N_EDGES = 320000
D_IN = 128
D_HID = 128
N_CLASSES = 64

NPAD = 10240
NC = 2
NS = 16
NW = NC * NS
EPAD = 327680
EPW = EPAD // NW
CH = 128
NCHUNK = EPW // CH
RPT = NPAD // NS

_MESH = plsc.VectorSubcoreMesh(
    core_axis_name="c", subcore_axis_name="s", num_cores=NC, num_subcores=NS
)

_F32 = jnp.float32


def _zv():
    return jnp.zeros((16,), _F32)


def _deg_body(src_hbm, dst_hbm, out_hbm, is0, is1, id0, id1, ones_v, zbuf,
              deg_s, deg_d, ss0, ss1, sd0, sd1):
    c = lax.axis_index("c")
    s = lax.axis_index("s")
    for j in range(CH // 16):
        ones_v[pl.ds(j * 16, 16)] = jnp.ones((16,), _F32)
    for j in range(RPT // 16):
        zbuf[pl.ds(j * 16, 16)] = _zv()
    zoff = pl.multiple_of(s * RPT, 8)
    pltpu.sync_copy(zbuf, deg_s.at[pl.ds(zoff, RPT)])
    pltpu.sync_copy(zbuf, deg_d.at[pl.ds(zoff, RPT)])
    plsc.subcore_barrier()
    ebase = pl.multiple_of((c * NS + s) * EPW, 8)

    pltpu.async_copy(src_hbm.at[pl.ds(ebase, CH)], is0, ss0)
    pltpu.async_copy(dst_hbm.at[pl.ds(ebase, CH)], id0, sd0)
    pltpu.async_copy(src_hbm.at[pl.ds(pl.multiple_of(ebase + CH, 8), CH)], is1, ss1)
    pltpu.async_copy(dst_hbm.at[pl.ds(pl.multiple_of(ebase + CH, 8), CH)], id1, sd1)

    def body(i, carry):
        a = 2 * i
        n0 = pl.multiple_of(ebase + ((a + 2) % NCHUNK) * CH, 8)
        n1 = pl.multiple_of(ebase + ((a + 3) % NCHUNK) * CH, 8)
        pltpu.make_async_copy(src_hbm.at[pl.ds(ebase, CH)], is0, ss0).wait()
        pltpu.sync_copy(ones_v, deg_s.at[is0], add=True)
        pltpu.async_copy(src_hbm.at[pl.ds(n0, CH)], is0, ss0)
        pltpu.make_async_copy(dst_hbm.at[pl.ds(ebase, CH)], id0, sd0).wait()
        pltpu.sync_copy(ones_v, deg_d.at[id0], add=True)
        pltpu.async_copy(dst_hbm.at[pl.ds(n0, CH)], id0, sd0)
        pltpu.make_async_copy(src_hbm.at[pl.ds(ebase, CH)], is1, ss1).wait()
        pltpu.sync_copy(ones_v, deg_s.at[is1], add=True)
        pltpu.async_copy(src_hbm.at[pl.ds(n1, CH)], is1, ss1)
        pltpu.make_async_copy(dst_hbm.at[pl.ds(ebase, CH)], id1, sd1).wait()
        pltpu.sync_copy(ones_v, deg_d.at[id1], add=True)
        pltpu.async_copy(dst_hbm.at[pl.ds(n1, CH)], id1, sd1)
        return carry

    lax.fori_loop(0, NCHUNK // 2, body, 0)
    pltpu.make_async_copy(src_hbm.at[pl.ds(ebase, CH)], is0, ss0).wait()
    pltpu.make_async_copy(dst_hbm.at[pl.ds(ebase, CH)], id0, sd0).wait()
    pltpu.make_async_copy(src_hbm.at[pl.ds(ebase, CH)], is1, ss1).wait()
    pltpu.make_async_copy(dst_hbm.at[pl.ds(ebase, CH)], id1, sd1).wait()
    plsc.subcore_barrier()
    pltpu.sync_copy(deg_s.at[pl.ds(zoff, RPT)], out_hbm.at[c, 0, pl.ds(zoff, RPT)])
    pltpu.sync_copy(deg_d.at[pl.ds(zoff, RPT)], out_hbm.at[c, 1, pl.ds(zoff, RPT)])


_deg_call = pl.kernel(
    _deg_body,
    out_type=jax.ShapeDtypeStruct((NC, 2, NPAD), _F32),
    mesh=_MESH,
    scratch_types=[
        pltpu.VMEM((CH,), jnp.int32),
        pltpu.VMEM((CH,), jnp.int32),
        pltpu.VMEM((CH,), jnp.int32),
        pltpu.VMEM((CH,), jnp.int32),
        pltpu.VMEM((CH,), _F32),
        pltpu.VMEM((RPT,), _F32),
        pltpu.VMEM_SHARED((NPAD,), _F32),
        pltpu.VMEM_SHARED((NPAD,), _F32),
        pltpu.SemaphoreType.DMA,
        pltpu.SemaphoreType.DMA,
        pltpu.SemaphoreType.DMA,
        pltpu.SemaphoreType.DMA,
    ],
)


def _make_agg(d):
    def _agg_body(p_hbm, src_hbm, dst_hbm, out_hbm, idx_sf,
                  idxb0, idxb1, rows0, rows1, agg_sh,
                  sem0, sem1, ism0, ism1):
        c = lax.axis_index("c")
        s = lax.axis_index("s")
        idxb = [idxb0, idxb1]
        rows = [rows0, rows1]
        sem = [sem0, sem1]
        ism = [ism0, ism1]
        DEPTH = 2

        def zbody(r, carry):
            for k in range(d // 16):
                rows0[r, pl.ds(k * 16, 16)] = _zv()
            return carry

        lax.fori_loop(0, CH, zbody, 0)
        for t in range(RPT // CH):
            roff = pl.multiple_of(s * RPT + t * CH, 8)
            pltpu.sync_copy(rows0, agg_sh.at[pl.ds(roff, CH)])

        ebase = pl.multiple_of((c * NS + s) * EPW, 8)
        pltpu.sync_copy(src_hbm.at[pl.ds(ebase, EPW)], idx_sf)
        plsc.subcore_barrier()

        for k in range(DEPTH):
            o = pl.multiple_of(k * CH, 8)
            pltpu.async_copy(p_hbm.at[idx_sf.at[pl.ds(o, CH)]], rows[k], sem[k])
            pltpu.async_copy(
                dst_hbm.at[pl.ds(pl.multiple_of(ebase + o, 8), CH)], idxb[k], ism[k])

        def body(i, carry):
            a = DEPTH * i
            for k in range(DEPTH):
                o = pl.multiple_of((a + k) * CH, 8)
                n = pl.multiple_of(((a + k + DEPTH) % NCHUNK) * CH, 8)
                pltpu.make_async_copy(
                    dst_hbm.at[pl.ds(ebase, CH)], idxb[k], ism[k]).wait()
                pltpu.make_async_copy(
                    p_hbm.at[idx_sf.at[pl.ds(o, CH)]], rows[k], sem[k]).wait()
                pltpu.sync_copy(rows[k], agg_sh.at[idxb[k]], add=True)
                pltpu.async_copy(p_hbm.at[idx_sf.at[pl.ds(n, CH)]], rows[k], sem[k])
                pltpu.async_copy(
                    dst_hbm.at[pl.ds(pl.multiple_of(ebase + n, 8), CH)], idxb[k], ism[k])
            return carry

        lax.fori_loop(0, NCHUNK // DEPTH, body, 0)
        for k in range(DEPTH):
            o = pl.multiple_of(k * CH, 8)
            pltpu.make_async_copy(p_hbm.at[idx_sf.at[pl.ds(o, CH)]], rows[k], sem[k]).wait()
            pltpu.make_async_copy(dst_hbm.at[pl.ds(ebase, CH)], idxb[k], ism[k]).wait()
        plsc.subcore_barrier()
        roff = pl.multiple_of(s * RPT, 8)
        pltpu.sync_copy(agg_sh.at[pl.ds(roff, RPT)], out_hbm.at[c, pl.ds(roff, RPT)])

    return pl.kernel(
        _agg_body,
        out_type=jax.ShapeDtypeStruct((NC, NPAD, d), _F32),
        mesh=_MESH,
        scratch_types=[
            pltpu.VMEM((EPW,), jnp.int32),
            pltpu.VMEM((CH,), jnp.int32),
            pltpu.VMEM((CH,), jnp.int32),
            pltpu.VMEM((CH, d), _F32),
            pltpu.VMEM((CH, d), _F32),
            pltpu.VMEM_SHARED((NPAD, d), _F32),
            pltpu.SemaphoreType.DMA,
            pltpu.SemaphoreType.DMA,
            pltpu.SemaphoreType.DMA,
            pltpu.SemaphoreType.DMA,
        ],
    )


_agg128 = _make_agg(D_HID)

BR = 1024


def _norm(d0, d1):
    deg = d0 + d1
    return lax.rsqrt(jnp.where(deg > 0.0, deg, 1.0))


def _scale_mm_body(x_ref, w_ref, d0_ref, d1_ref, o_ref):
    ns = _norm(d0_ref[...], d1_ref[...])
    o_ref[...] = jnp.dot(x_ref[...], w_ref[...], preferred_element_type=_F32) * ns


def _mid_body(p_ref, dd0_ref, dd1_ref, b1_ref, w2_ref, ds0_ref, ds1_ref, o_ref):
    nd = _norm(dd0_ref[...], dd1_ref[...])
    h = jnp.maximum((p_ref[0] + p_ref[1]) * nd + b1_ref[...], 0.0)
    ns = _norm(ds0_ref[...], ds1_ref[...])
    o_ref[...] = jnp.dot(h, w2_ref[...], preferred_element_type=_F32) * ns


def _fin_body(p_ref, dd0_ref, dd1_ref, b2_ref, o_ref):
    nd = _norm(dd0_ref[...], dd1_ref[...])
    o_ref[...] = (p_ref[0, :, :N_CLASSES] + p_ref[1, :, :N_CLASSES]) * nd + b2_ref[...]


def _col_spec():
    return pl.BlockSpec((BR, 1), lambda i: (i, 0))


def _row_spec(d):
    return pl.BlockSpec((BR, d), lambda i: (i, 0))


def _full_spec(a, b):
    return pl.BlockSpec((a, b), lambda i: (0, 0))


def _scale_mm(x, w, d0, d1):
    d = w.shape[1]
    return pl.pallas_call(
        _scale_mm_body,
        grid=(NPAD // BR,),
        in_specs=[_row_spec(x.shape[1]), _full_spec(*w.shape), _col_spec(), _col_spec()],
        out_specs=_row_spec(d),
        out_shape=jax.ShapeDtypeStruct((NPAD, d), _F32),
    )(x, w, d0, d1)


def _parts_spec(d, br):
    return pl.BlockSpec((NC, br, d), lambda i: (0, i, 0))


def _mid(parts, dd0, dd1, b1, w2, ds0, ds1):
    dout = w2.shape[1]
    return pl.pallas_call(
        _mid_body,
        grid=(NPAD // BR,),
        in_specs=[
            _parts_spec(D_HID, BR), _col_spec(), _col_spec(),
            _full_spec(1, D_HID), _full_spec(D_HID, dout), _col_spec(), _col_spec(),
        ],
        out_specs=_row_spec(dout),
        out_shape=jax.ShapeDtypeStruct((NPAD, dout), _F32),
    )(parts, dd0, dd1, b1, w2, ds0, ds1)


BRF = 1000


def _fin(parts, dd0, dd1, b2):
    return pl.pallas_call(
        _fin_body,
        grid=(N_NODES // BRF,),
        in_specs=[
            _parts_spec(D_HID, BRF),
            pl.BlockSpec((BRF, 1), lambda i: (i, 0)),
            pl.BlockSpec((BRF, 1), lambda i: (i, 0)),
            _full_spec(1, N_CLASSES),
        ],
        out_specs=pl.BlockSpec((BRF, N_CLASSES), lambda i: (i, 0)),
        out_shape=jax.ShapeDtypeStruct((N_NODES, N_CLASSES), _F32),
    )(parts, dd0, dd1, b2)


def kernel(inputs, edge_index, W1, b1, W2, b2):
    x = jnp.pad(inputs, ((0, NPAD - N_NODES), (0, 0)))
    npd = EPAD - N_EDGES
    pad = N_NODES + jax.lax.iota(jnp.int32, npd) % (NPAD - N_NODES)
    src = jnp.concatenate([edge_index[0], pad])
    dst = jnp.concatenate([edge_index[1], pad])

    deg = _deg_call(src, dst)
    d_s = deg[:, 0, :].reshape(NC, NPAD, 1)
    d_d = deg[:, 1, :].reshape(NC, NPAD, 1)

    p1 = _scale_mm(x, W1, d_s[0], d_s[1])
    parts1 = _agg128(p1, src, dst)
    w2p = jnp.pad(W2, ((0, 0), (0, D_HID - N_CLASSES)))
    p2 = _mid(parts1, d_d[0], d_d[1],
              b1.reshape(1, D_HID), w2p, d_s[0], d_s[1])
    parts2 = _agg128(p2, src, dst)
    return _fin(parts2, d_d[0], d_d[1], b2.reshape(1, N_CLASSES))
